# baseline (speedup 1.0000x reference)
import functools

import jax
import jax.numpy as jnp
from jax import lax
from jax.experimental import pallas as pl
from jax.experimental.pallas import tpu as pltpu
from jax.experimental.pallas import tpu_sc as plsc

N = 10000
E = 320000
D_IN = 128
D_HID = 128
D_OUT = 64

NC = 2
NS = 16
NW = NC * NS
K = 128
NBUF = 5

J = 80
E_PAD = NW * J * K

N_PAD = 10240
STRIPE = N_PAD // NS
DUMMY = N

_mesh = plsc.VectorSubcoreMesh(
    core_axis_name="c", subcore_axis_name="s", num_cores=NC, num_subcores=NS)


@functools.partial(
    pl.kernel,
    out_type=jax.ShapeDtypeStruct((NW, N_PAD), jnp.float32),
    mesh=_mesh,
    compiler_params=pltpu.CompilerParams(needs_layout_passes=False),
    scratch_types=[
        pltpu.VMEM((J, K), jnp.int32),
        pltpu.VMEM((N_PAD,), jnp.float32),
    ],
)
def _deg_kernel(dst_hbm, out_hbm, idx_v, hist_v):
    c = lax.axis_index("c")
    s = lax.axis_index("s")
    w = c * NS + s
    pltpu.sync_copy(dst_hbm.at[w], idx_v)

    zeros16 = jnp.zeros((16,), jnp.float32)

    def zero_body(i, carry):
        hist_v[pl.ds(i * 16, 16)] = zeros16
        return carry

    lax.fori_loop(0, N_PAD // 16, zero_body, 0)

    ones16 = jnp.ones((16,), jnp.float32)

    def hist_body(i, carry):
        j = i // (K // 16)
        k = i % (K // 16)
        idx = idx_v[j, pl.ds(k * 16, 16)]
        plsc.addupdate_scatter(hist_v, [idx], ones16)
        return carry

    lax.fori_loop(0, (J * K) // 16, hist_body, 0)
    pltpu.sync_copy(hist_v, out_hbm.at[w])


def _make_scatter(D):
    @functools.partial(
        pl.kernel,
        out_type=jax.ShapeDtypeStruct((NC, N_PAD, D), jnp.float32),
        mesh=_mesh,
        compiler_params=pltpu.CompilerParams(use_tc_tiling_on_sc=False),
        scratch_types=[
            pltpu.VMEM((J, K), jnp.int32),
            pltpu.VMEM((J, K), jnp.int32),
            pltpu.VMEM((NBUF, K, D), jnp.float32),
            pltpu.VMEM_SHARED((N_PAD, D), jnp.float32),
            [pltpu.SemaphoreType.DMA] * NBUF,
        ],
    )
    def _scatter_kernel(g_hbm, src_hbm, dst_hbm, out_hbm,
                        src_v, dst_v, rows_v, acc_sh, gsems):
        c = lax.axis_index("c")
        s = lax.axis_index("s")
        w = c * NS + s
        pltpu.sync_copy(src_hbm.at[w], src_v)
        pltpu.sync_copy(dst_hbm.at[w], dst_v)

        def _gather(j, b):
            pltpu.async_copy(g_hbm.at[src_v.at[j]], rows_v.at[b], gsems[b])

        def _gather_wait(j, b):
            pltpu.make_async_copy(
                g_hbm.at[src_v.at[j]], rows_v.at[b], gsems[b]).wait()

        def _scatter_sync(j, b):
            pltpu.sync_copy(rows_v.at[b], acc_sh.at[dst_v.at[j]], add=True)

        for b in range(NBUF - 1):
            _gather(b, b)

        zeros16 = jnp.zeros((16,), jnp.float32)

        def zrow(i, carry):
            r = i // (D // 16)
            l = i % (D // 16)
            rows_v[NBUF - 1, r, pl.ds(l * 16, 16)] = zeros16
            return carry

        lax.fori_loop(0, (K * D) // 16, zrow, 0)
        for p in range(STRIPE // K):
            pltpu.sync_copy(rows_v.at[NBUF - 1],
                            acc_sh.at[pl.ds(s * STRIPE + p * K, K)])
        plsc.subcore_barrier()
        _gather(NBUF - 1, NBUF - 1)

        ngrp = J // NBUF

        def body(g, carry):
            for b in range(NBUF):
                j = g * NBUF + b
                _gather_wait(j, b)
                _scatter_sync(j, b)
                _gather(j + NBUF, b)
            return carry

        lax.fori_loop(0, ngrp - 1, body, 0)
        for b in range(NBUF):
            jt = (ngrp - 1) * NBUF + b
            _gather_wait(jt, b)
            _scatter_sync(jt, b)

        plsc.subcore_barrier()
        pltpu.sync_copy(acc_sh.at[pl.ds(s * STRIPE, STRIPE)],
                        out_hbm.at[c, pl.ds(s * STRIPE, STRIPE)])

    return _scatter_kernel


_scatter64 = _make_scatter(D_OUT)


_R = 1024


def _dinv_of(hist_blk):
    deg = jnp.sum(hist_blk, axis=1, keepdims=True) + 1.0
    return lax.rsqrt(deg)


def _tc1_body(hist_ref, x_ref, w1_ref, g1a_ref, g1b_ref):
    dinv = _dinv_of(hist_ref[...])
    g1 = jnp.dot(x_ref[...], w1_ref[...],
                 preferred_element_type=jnp.float32) * dinv
    g1a_ref[...] = g1[:, :D_OUT]
    g1b_ref[...] = g1[:, D_OUT:]


def _tc2_body(hist_ref, acca_ref, accb_ref, g1a_ref, g1b_ref,
              b1_ref, w2a_ref, w2b_ref, g2_ref):
    dinv = _dinv_of(hist_ref[...])
    ha = jnp.maximum((acca_ref[0] + acca_ref[1] + g1a_ref[...]) * dinv
                     + b1_ref[:, :D_OUT], 0.0)
    hb = jnp.maximum((accb_ref[0] + accb_ref[1] + g1b_ref[...]) * dinv
                     + b1_ref[:, D_OUT:], 0.0)
    g2 = (jnp.dot(ha, w2a_ref[...], preferred_element_type=jnp.float32)
          + jnp.dot(hb, w2b_ref[...], preferred_element_type=jnp.float32))
    g2_ref[...] = g2 * dinv


def _tc3_body(hist_ref, acc_ref, g2_ref, b2_ref, out_ref):
    dinv = _dinv_of(hist_ref[...])
    a = acc_ref[0] + acc_ref[1] + g2_ref[...]
    out_ref[...] = jnp.maximum(a * dinv + b2_ref[...], 0.0)


def _row_spec(d):
    return pl.BlockSpec((_R, d), lambda i: (i, 0))


def _acc_spec(d):
    return pl.BlockSpec((NC, _R, d), lambda i: (0, i, 0))


def _full_spec(r, d):
    return pl.BlockSpec((r, d), lambda i: (0, 0))


_GRID = N_PAD // _R

_tc1 = pl.pallas_call(
    _tc1_body,
    grid=(_GRID,),
    in_specs=[_row_spec(NW), _row_spec(D_IN), _full_spec(D_IN, D_HID)],
    out_specs=[_row_spec(D_OUT), _row_spec(D_OUT)],
    out_shape=[jax.ShapeDtypeStruct((N_PAD, D_OUT), jnp.float32),
               jax.ShapeDtypeStruct((N_PAD, D_OUT), jnp.float32)],
)

_tc2 = pl.pallas_call(
    _tc2_body,
    grid=(_GRID,),
    in_specs=[_row_spec(NW), _acc_spec(D_OUT), _acc_spec(D_OUT),
              _row_spec(D_OUT), _row_spec(D_OUT),
              _full_spec(1, D_HID), _full_spec(D_OUT, D_OUT),
              _full_spec(D_OUT, D_OUT)],
    out_specs=_row_spec(D_OUT),
    out_shape=jax.ShapeDtypeStruct((N_PAD, D_OUT), jnp.float32),
)

_tc3 = pl.pallas_call(
    _tc3_body,
    grid=(_GRID,),
    in_specs=[_row_spec(NW), _acc_spec(D_OUT), _row_spec(D_OUT),
              _full_spec(1, D_OUT)],
    out_specs=_row_spec(D_OUT),
    out_shape=jax.ShapeDtypeStruct((N_PAD, D_OUT), jnp.float32),
)


def _edge_blocks(idx_flat):
    fill = N + (jnp.arange(E_PAD - E, dtype=jnp.int32) % (N_PAD - N))
    return jnp.concatenate([idx_flat, fill]).reshape(NW, J, K)


def kernel(x, edge_index, W1, b1, W2, b2):
    x_pad = jnp.zeros((N_PAD, D_IN), x.dtype).at[:N].set(x)
    src = _edge_blocks(edge_index[0])
    dst = _edge_blocks(edge_index[1])

    hist = _deg_kernel(dst)
    hist_t = hist.T

    g1a, g1b = _tc1(hist_t, x_pad, W1)
    acc1a = _scatter64(g1a, src, dst)
    acc1b = _scatter64(g1b, src, dst)
    g2 = _tc2(hist_t, acc1a, acc1b, g1a, g1b, b1.reshape(1, D_HID),
              W2[:D_OUT], W2[D_OUT:])
    acc2 = _scatter64(g2, src, dst)
    out = _tc3(hist_t, acc2, g2, b2.reshape(1, D_OUT))
    return out[:N]


# speedup vs baseline: 1.0074x; 1.0074x over previous
import functools

import jax
import jax.numpy as jnp
from jax import lax
from jax.experimental import pallas as pl
from jax.experimental.pallas import tpu as pltpu
from jax.experimental.pallas import tpu_sc as plsc

N = 10000
E = 320000
D_IN = 128
D_HID = 128
D_OUT = 64

NC = 2
NS = 16
NW = NC * NS
K = 128
NBUF = 4

J = 80
E_PAD = NW * J * K

N_PAD = 10240
STRIPE = N_PAD // NS
DUMMY = N

_mesh = plsc.VectorSubcoreMesh(
    core_axis_name="c", subcore_axis_name="s", num_cores=NC, num_subcores=NS)


@functools.partial(
    pl.kernel,
    out_type=jax.ShapeDtypeStruct((NW, N_PAD), jnp.float32),
    mesh=_mesh,
    compiler_params=pltpu.CompilerParams(needs_layout_passes=False),
    scratch_types=[
        pltpu.VMEM((J, K), jnp.int32),
        pltpu.VMEM((N_PAD,), jnp.float32),
    ],
)
def _deg_kernel(dst_hbm, out_hbm, idx_v, hist_v):
    c = lax.axis_index("c")
    s = lax.axis_index("s")
    w = c * NS + s
    pltpu.sync_copy(dst_hbm.at[w], idx_v)

    zeros16 = jnp.zeros((16,), jnp.float32)

    def zero_body(i, carry):
        hist_v[pl.ds(i * 16, 16)] = zeros16
        return carry

    lax.fori_loop(0, N_PAD // 16, zero_body, 0)

    ones16 = jnp.ones((16,), jnp.float32)

    def hist_body(i, carry):
        j = i // (K // 16)
        k = i % (K // 16)
        idx = idx_v[j, pl.ds(k * 16, 16)]
        plsc.addupdate_scatter(hist_v, [idx], ones16)
        return carry

    lax.fori_loop(0, (J * K) // 16, hist_body, 0)
    pltpu.sync_copy(hist_v, out_hbm.at[w])


def _make_scatter(D, npass):
    @functools.partial(
        pl.kernel,
        out_type=[jax.ShapeDtypeStruct((NC, N_PAD, D), jnp.float32)] * npass,
        mesh=_mesh,
        compiler_params=pltpu.CompilerParams(use_tc_tiling_on_sc=False),
        scratch_types=[
            pltpu.VMEM((J, K), jnp.int32),
            pltpu.VMEM((J, K), jnp.int32),
            pltpu.VMEM((NBUF, K, D), jnp.float32),
            pltpu.VMEM((K, D), jnp.float32),
            pltpu.VMEM_SHARED((N_PAD, D), jnp.float32),
            [pltpu.SemaphoreType.DMA] * NBUF,
        ],
    )
    def _scatter_kernel(*args):
        g_hbms = args[:npass]
        src_hbm, dst_hbm = args[npass], args[npass + 1]
        out_hbms = args[npass + 2:2 * npass + 2]
        src_v, dst_v, rows_v, zero_v, acc_sh, gsems = args[2 * npass + 2:]
        c = lax.axis_index("c")
        s = lax.axis_index("s")
        w = c * NS + s
        pltpu.sync_copy(src_hbm.at[w], src_v)
        pltpu.sync_copy(dst_hbm.at[w], dst_v)

        def _gather(g_hbm, j, b):
            pltpu.async_copy(g_hbm.at[src_v.at[j]], rows_v.at[b], gsems[b])

        def _gather_wait(g_hbm, j, b):
            pltpu.make_async_copy(
                g_hbm.at[src_v.at[j]], rows_v.at[b], gsems[b]).wait()

        def _scatter_sync(j, b):
            pltpu.sync_copy(rows_v.at[b], acc_sh.at[dst_v.at[j]], add=True)

        def _zero_stripe():
            for p in range(STRIPE // K):
                pltpu.sync_copy(zero_v,
                                acc_sh.at[pl.ds(s * STRIPE + p * K, K)])

        zeros16 = jnp.zeros((16,), jnp.float32)

        def zrow(i, carry):
            r = i // (D // 16)
            l = i % (D // 16)
            zero_v[r, pl.ds(l * 16, 16)] = zeros16
            return carry

        for b in range(NBUF):
            _gather(g_hbms[0], b, b)
        lax.fori_loop(0, (K * D) // 16, zrow, 0)
        _zero_stripe()
        plsc.subcore_barrier()

        ngrp = J // NBUF
        for p in range(npass):
            g_hbm = g_hbms[p]
            nxt = g_hbms[p + 1] if p + 1 < npass else None

            def body(g, carry, g_hbm=g_hbm):
                for b in range(NBUF):
                    j = g * NBUF + b
                    _gather_wait(g_hbm, j, b)
                    _scatter_sync(j, b)
                    _gather(g_hbm, j + NBUF, b)
                return carry

            lax.fori_loop(0, ngrp - 1, body, 0)
            for b in range(NBUF):
                jt = (ngrp - 1) * NBUF + b
                _gather_wait(g_hbm, jt, b)
                _scatter_sync(jt, b)
                if nxt is not None:
                    _gather(nxt, b, b)
            plsc.subcore_barrier()
            pltpu.sync_copy(acc_sh.at[pl.ds(s * STRIPE, STRIPE)],
                            out_hbms[p].at[c, pl.ds(s * STRIPE, STRIPE)])
            if nxt is not None:
                _zero_stripe()
                plsc.subcore_barrier()

    return _scatter_kernel


_scatter_l1 = _make_scatter(D_OUT, 2)
_scatter_l2 = _make_scatter(D_OUT, 1)


_R = 1024


def _dinv_of(hist_blk):
    deg = jnp.sum(hist_blk, axis=1, keepdims=True) + 1.0
    return lax.rsqrt(deg)


def _tc1_body(hist_ref, x_ref, w1_ref, g1a_ref, g1b_ref, dinv_ref):
    dinv = _dinv_of(hist_ref[...])
    g1 = jnp.dot(x_ref[...], w1_ref[...],
                 preferred_element_type=jnp.float32) * dinv
    g1a_ref[...] = g1[:, :D_OUT]
    g1b_ref[...] = g1[:, D_OUT:]
    dinv_ref[...] = dinv


def _tc2_body(dinv_ref, acca_ref, accb_ref, g1a_ref, g1b_ref,
              b1_ref, w2a_ref, w2b_ref, g2_ref):
    dinv = dinv_ref[...]
    ha = jnp.maximum((acca_ref[0] + acca_ref[1] + g1a_ref[...]) * dinv
                     + b1_ref[:, :D_OUT], 0.0)
    hb = jnp.maximum((accb_ref[0] + accb_ref[1] + g1b_ref[...]) * dinv
                     + b1_ref[:, D_OUT:], 0.0)
    g2 = (jnp.dot(ha, w2a_ref[...], preferred_element_type=jnp.float32)
          + jnp.dot(hb, w2b_ref[...], preferred_element_type=jnp.float32))
    g2_ref[...] = g2 * dinv


def _tc3_body(dinv_ref, acc_ref, g2_ref, b2_ref, out_ref):
    dinv = dinv_ref[...]
    a = acc_ref[0] + acc_ref[1] + g2_ref[...]
    out_ref[...] = jnp.maximum(a * dinv + b2_ref[...], 0.0)


def _row_spec(d):
    return pl.BlockSpec((_R, d), lambda i: (i, 0))


def _acc_spec(d):
    return pl.BlockSpec((NC, _R, d), lambda i: (0, i, 0))


def _full_spec(r, d):
    return pl.BlockSpec((r, d), lambda i: (0, 0))


_GRID = N_PAD // _R

_tc1 = pl.pallas_call(
    _tc1_body,
    grid=(_GRID,),
    in_specs=[_row_spec(NW), _row_spec(D_IN), _full_spec(D_IN, D_HID)],
    out_specs=[_row_spec(D_OUT), _row_spec(D_OUT), _row_spec(1)],
    out_shape=[jax.ShapeDtypeStruct((N_PAD, D_OUT), jnp.float32),
               jax.ShapeDtypeStruct((N_PAD, D_OUT), jnp.float32),
               jax.ShapeDtypeStruct((N_PAD, 1), jnp.float32)],
)

_tc2 = pl.pallas_call(
    _tc2_body,
    grid=(_GRID,),
    in_specs=[_row_spec(1), _acc_spec(D_OUT), _acc_spec(D_OUT),
              _row_spec(D_OUT), _row_spec(D_OUT),
              _full_spec(1, D_HID), _full_spec(D_OUT, D_OUT),
              _full_spec(D_OUT, D_OUT)],
    out_specs=_row_spec(D_OUT),
    out_shape=jax.ShapeDtypeStruct((N_PAD, D_OUT), jnp.float32),
)

_tc3 = pl.pallas_call(
    _tc3_body,
    grid=(_GRID,),
    in_specs=[_row_spec(1), _acc_spec(D_OUT), _row_spec(D_OUT),
              _full_spec(1, D_OUT)],
    out_specs=_row_spec(D_OUT),
    out_shape=jax.ShapeDtypeStruct((N, D_OUT), jnp.float32),
)


def _edge_blocks(idx_flat):
    fill = N + (jnp.arange(E_PAD - E, dtype=jnp.int32) % (N_PAD - N))
    return jnp.concatenate([idx_flat, fill]).reshape(NW, J, K)


def kernel(x, edge_index, W1, b1, W2, b2):
    x_pad = jnp.zeros((N_PAD, D_IN), x.dtype).at[:N].set(x)
    src = _edge_blocks(edge_index[0])
    dst = _edge_blocks(edge_index[1])

    hist = _deg_kernel(dst)
    hist_t = hist.T

    g1a, g1b, dinv = _tc1(hist_t, x_pad, W1)
    acc1a, acc1b = _scatter_l1(g1a, g1b, src, dst)
    g2 = _tc2(dinv, acc1a, acc1b, g1a, g1b, b1.reshape(1, D_HID),
              W2[:D_OUT], W2[D_OUT:])
    acc2, = _scatter_l2(g2, src, dst)
    return _tc3(dinv, acc2, g2, b2.reshape(1, D_OUT))


# speedup vs baseline: 1.1687x; 1.1601x over previous
import functools

import jax
import jax.numpy as jnp
from jax import lax
from jax.experimental import pallas as pl
from jax.experimental.pallas import tpu as pltpu
from jax.experimental.pallas import tpu_sc as plsc

N = 10000
E = 320000
D_IN = 128
D_HID = 128
D_OUT = 64

NC = 2
NS = 16
NW = NC * NS
K = 128
NBUF = 4

J = 80
E_PAD = NW * J * K

N_PAD = 10240
STRIPE = N_PAD // NS
DUMMY = N

_mesh = plsc.VectorSubcoreMesh(
    core_axis_name="c", subcore_axis_name="s", num_cores=NC, num_subcores=NS)


@functools.partial(
    pl.kernel,
    out_type=jax.ShapeDtypeStruct((NW, N_PAD), jnp.float32),
    mesh=_mesh,
    compiler_params=pltpu.CompilerParams(needs_layout_passes=False),
    scratch_types=[
        pltpu.VMEM((J, K), jnp.int32),
        pltpu.VMEM((N_PAD,), jnp.float32),
    ],
)
def _deg_kernel(dst_hbm, out_hbm, idx_v, hist_v):
    c = lax.axis_index("c")
    s = lax.axis_index("s")
    w = c * NS + s
    pltpu.sync_copy(dst_hbm.at[w], idx_v)

    zeros16 = jnp.zeros((16,), jnp.float32)

    def zero_body(i, carry):
        hist_v[pl.ds(i * 16, 16)] = zeros16
        return carry

    lax.fori_loop(0, N_PAD // 16, zero_body, 0)

    ones16 = jnp.ones((16,), jnp.float32)

    def hist_body(i, carry):
        j = i // (K // 16)
        k = i % (K // 16)
        idx = idx_v[j, pl.ds(k * 16, 16)]
        plsc.addupdate_scatter(hist_v, [idx], ones16)
        return carry

    lax.fori_loop(0, (J * K) // 16, hist_body, 0)
    pltpu.sync_copy(hist_v, out_hbm.at[w])


def _make_scatter(D, npass):
    @functools.partial(
        pl.kernel,
        out_type=[jax.ShapeDtypeStruct((NC, N_PAD, D), jnp.float32)] * npass,
        mesh=_mesh,
        compiler_params=pltpu.CompilerParams(use_tc_tiling_on_sc=False),
        scratch_types=[
            pltpu.VMEM((J, K), jnp.int32),
            pltpu.VMEM((J, K), jnp.int32),
            pltpu.VMEM((NBUF, K, D), jnp.float32),
            pltpu.VMEM((K, D), jnp.float32),
            pltpu.VMEM_SHARED((N_PAD, D), jnp.float32),
            [pltpu.SemaphoreType.DMA] * NBUF,
        ],
    )
    def _scatter_kernel(*args):
        g_hbms = args[:npass]
        src_hbm, dst_hbm = args[npass], args[npass + 1]
        out_hbms = args[npass + 2:2 * npass + 2]
        src_v, dst_v, rows_v, zero_v, acc_sh, gsems = args[2 * npass + 2:]
        c = lax.axis_index("c")
        s = lax.axis_index("s")
        w = c * NS + s
        pltpu.sync_copy(src_hbm.at[w], src_v)
        pltpu.sync_copy(dst_hbm.at[w], dst_v)

        def _gather(g_hbm, j, b):
            pltpu.async_copy(g_hbm.at[src_v.at[j]], rows_v.at[b], gsems[b])

        def _gather_wait(g_hbm, j, b):
            pltpu.make_async_copy(
                g_hbm.at[src_v.at[j]], rows_v.at[b], gsems[b]).wait()

        def _scatter_sync(j, b):
            pltpu.sync_copy(rows_v.at[b], acc_sh.at[dst_v.at[j]], add=True)

        def _zero_stripe():
            for p in range(STRIPE // K):
                pltpu.sync_copy(zero_v,
                                acc_sh.at[pl.ds(s * STRIPE + p * K, K)])

        zeros16 = jnp.zeros((16,), jnp.float32)

        def zrow(i, carry):
            r = i // (D // 16)
            l = i % (D // 16)
            zero_v[r, pl.ds(l * 16, 16)] = zeros16
            return carry

        for b in range(NBUF):
            _gather(g_hbms[0], b, b)
        lax.fori_loop(0, (K * D) // 16, zrow, 0)
        _zero_stripe()
        plsc.subcore_barrier()

        ngrp = J // NBUF
        for p in range(npass):
            g_hbm = g_hbms[p]
            nxt = g_hbms[p + 1] if p + 1 < npass else None

            def body(g, carry, g_hbm=g_hbm):
                for b in range(NBUF):
                    j = g * NBUF + b
                    _gather_wait(g_hbm, j, b)
                    _scatter_sync(j, b)
                    _gather(g_hbm, j + NBUF, b)
                return carry

            lax.fori_loop(0, ngrp - 1, body, 0)
            for b in range(NBUF):
                jt = (ngrp - 1) * NBUF + b
                _gather_wait(g_hbm, jt, b)
                _scatter_sync(jt, b)
                if nxt is not None:
                    _gather(nxt, b, b)
            plsc.subcore_barrier()
            pltpu.sync_copy(acc_sh.at[pl.ds(s * STRIPE, STRIPE)],
                            out_hbms[p].at[c, pl.ds(s * STRIPE, STRIPE)])
            if nxt is not None:
                _zero_stripe()
                plsc.subcore_barrier()

    return _scatter_kernel


_scatter_l1 = _make_scatter(D_OUT, 2)
_scatter_l2 = _make_scatter(D_OUT, 1)


_R = 1024


def _dinv_of(hist_blk):
    deg = jnp.sum(hist_blk, axis=1, keepdims=True) + 1.0
    return lax.rsqrt(deg)


def _tc1_body(hist_ref, x_ref, w1_ref, g1a_ref, g1b_ref, dinv_ref):
    dinv = _dinv_of(hist_ref[...])
    g1 = jnp.dot(x_ref[...], w1_ref[...],
                 preferred_element_type=jnp.float32) * dinv
    g1a_ref[...] = g1[:, :D_OUT]
    g1b_ref[...] = g1[:, D_OUT:]
    dinv_ref[...] = dinv


def _tc2_body(dinv_ref, acca_ref, accb_ref, g1a_ref, g1b_ref,
              b1a_ref, b1b_ref, w2a_ref, w2b_ref, g2_ref):
    dinv = dinv_ref[...]
    ha = jnp.maximum((acca_ref[0] + acca_ref[1] + g1a_ref[...]) * dinv
                     + b1a_ref[...], 0.0)
    hb = jnp.maximum((accb_ref[0] + accb_ref[1] + g1b_ref[...]) * dinv
                     + b1b_ref[...], 0.0)
    g2 = (jnp.dot(ha, w2a_ref[...], preferred_element_type=jnp.float32)
          + jnp.dot(hb, w2b_ref[...], preferred_element_type=jnp.float32))
    g2_ref[...] = g2 * dinv


def _tc3_body(dinv_ref, acc_ref, g2_ref, b2_ref, out_ref):
    dinv = dinv_ref[...]
    a = acc_ref[0] + acc_ref[1] + g2_ref[...]
    out_ref[...] = jnp.maximum(a * dinv + b2_ref[...], 0.0)


def _row_spec(d):
    return pl.BlockSpec((_R, d), lambda i: (i, 0))


def _acc_spec(d):
    return pl.BlockSpec((NC, _R, d), lambda i: (0, i, 0))


def _full_spec(r, d):
    return pl.BlockSpec((r, d), lambda i: (0, 0))


_GRID = N_PAD // _R

_tc1 = pl.pallas_call(
    _tc1_body,
    grid=(_GRID,),
    in_specs=[_row_spec(NW), _row_spec(D_IN), _full_spec(D_IN, D_HID)],
    out_specs=[_row_spec(D_OUT), _row_spec(D_OUT), _row_spec(1)],
    out_shape=[jax.ShapeDtypeStruct((N_PAD, D_OUT), jnp.float32),
               jax.ShapeDtypeStruct((N_PAD, D_OUT), jnp.float32),
               jax.ShapeDtypeStruct((N_PAD, 1), jnp.float32)],
)

_PK = N_PAD // 2
_PKGRID = _PK // _R

_tc2 = pl.pallas_call(
    _tc2_body,
    grid=(_PKGRID,),
    in_specs=[_row_spec(D_HID), _acc_spec(D_HID), _acc_spec(D_HID),
              _row_spec(D_HID), _row_spec(D_HID),
              _full_spec(1, D_HID), _full_spec(1, D_HID),
              _full_spec(D_HID, D_HID), _full_spec(D_HID, D_HID)],
    out_specs=_row_spec(D_HID),
    out_shape=jax.ShapeDtypeStruct((_PK, D_HID), jnp.float32),
)

_tc3 = pl.pallas_call(
    _tc3_body,
    grid=(_PKGRID,),
    in_specs=[_row_spec(D_HID), _acc_spec(D_HID), _row_spec(D_HID),
              _full_spec(1, D_HID)],
    out_specs=_row_spec(D_HID),
    out_shape=jax.ShapeDtypeStruct((N // 2, D_HID), jnp.float32),
)


def _edge_blocks(idx_flat):
    fill = N + (jnp.arange(E_PAD - E, dtype=jnp.int32) % (N_PAD - N))
    return jnp.concatenate([idx_flat, fill]).reshape(NW, J, K)


def kernel(x, edge_index, W1, b1, W2, b2):
    x_pad = jnp.zeros((N_PAD, D_IN), x.dtype).at[:N].set(x)
    src = _edge_blocks(edge_index[0])
    dst = _edge_blocks(edge_index[1])

    hist = _deg_kernel(dst)
    hist_t = hist.T

    g1a, g1b, dinv = _tc1(hist_t, x_pad, W1)
    acc1a, acc1b = _scatter_l1(g1a, g1b, src, dst)

    pk = lambda a: a.reshape(_PK, D_HID)
    pk2 = lambda a: a.reshape(NC, _PK, D_HID)
    dinv_pk = jnp.repeat(dinv[:, 0], D_OUT).reshape(_PK, D_HID)
    zblk = jnp.zeros((D_OUT, D_OUT), jnp.float32)
    w2a_bd = jnp.block([[W2[:D_OUT], zblk], [zblk, W2[:D_OUT]]])
    w2b_bd = jnp.block([[W2[D_OUT:], zblk], [zblk, W2[D_OUT:]]])
    b1a = jnp.tile(b1[:D_OUT], 2).reshape(1, D_HID)
    b1b = jnp.tile(b1[D_OUT:], 2).reshape(1, D_HID)
    b2pk = jnp.tile(b2, 2).reshape(1, D_HID)

    g2_pk = _tc2(dinv_pk, pk2(acc1a), pk2(acc1b), pk(g1a), pk(g1b),
                 b1a, b1b, w2a_bd, w2b_bd)
    g2 = g2_pk.reshape(N_PAD, D_OUT)
    acc2, = _scatter_l2(g2, src, dst)
    out_pk = _tc3(dinv_pk, pk2(acc2), g2_pk, b2pk)
    return out_pk.reshape(N, D_OUT)


# speedup vs baseline: 1.1687x; 1.0000x over previous
import functools

import jax
import jax.numpy as jnp
from jax import lax
from jax.experimental import pallas as pl
from jax.experimental.pallas import tpu as pltpu
from jax.experimental.pallas import tpu_sc as plsc

N = 10000
E = 320000
D_IN = 128
D_HID = 128
D_OUT = 64

NC = 2
NS = 16
NW = NC * NS
K = 128
NBUF = 4

J = 80
E_PAD = NW * J * K

N_PAD = 10240
STRIPE = N_PAD // NS
DUMMY = N

_mesh = plsc.VectorSubcoreMesh(
    core_axis_name="c", subcore_axis_name="s", num_cores=NC, num_subcores=NS)


@functools.partial(
    pl.kernel,
    out_type=jax.ShapeDtypeStruct((NW, N_PAD), jnp.float32),
    mesh=_mesh,
    compiler_params=pltpu.CompilerParams(needs_layout_passes=False),
    scratch_types=[
        pltpu.VMEM((J, K), jnp.int32),
        pltpu.VMEM((N_PAD,), jnp.float32),
    ],
)
def _deg_kernel(dst_hbm, out_hbm, idx_v, hist_v):
    c = lax.axis_index("c")
    s = lax.axis_index("s")
    w = c * NS + s
    pltpu.sync_copy(dst_hbm.at[w], idx_v)

    zeros16 = jnp.zeros((16,), jnp.float32)

    def zero_body(i, carry):
        hist_v[pl.ds(i * 16, 16)] = zeros16
        return carry

    lax.fori_loop(0, N_PAD // 16, zero_body, 0)

    ones16 = jnp.ones((16,), jnp.float32)

    def hist_body(i, carry):
        j = i // (K // 16)
        k = i % (K // 16)
        idx = idx_v[j, pl.ds(k * 16, 16)]
        plsc.addupdate_scatter(hist_v, [idx], ones16)
        return carry

    lax.fori_loop(0, (J * K) // 16, hist_body, 0)
    pltpu.sync_copy(hist_v, out_hbm.at[w])


def _make_scatter(D, npass):
    @functools.partial(
        pl.kernel,
        out_type=[jax.ShapeDtypeStruct((NC, N_PAD, D), jnp.float32)] * npass,
        mesh=_mesh,
        compiler_params=pltpu.CompilerParams(use_tc_tiling_on_sc=False),
        scratch_types=[
            pltpu.VMEM((J, K), jnp.int32),
            pltpu.VMEM((J, K), jnp.int32),
            pltpu.VMEM((NBUF, K, D), jnp.float32),
            pltpu.VMEM((K, D), jnp.float32),
            pltpu.VMEM_SHARED((N_PAD, D), jnp.float32),
            [pltpu.SemaphoreType.DMA] * NBUF,
        ],
    )
    def _scatter_kernel(*args):
        g_hbms = args[:npass]
        src_hbm, dst_hbm = args[npass], args[npass + 1]
        out_hbms = args[npass + 2:2 * npass + 2]
        src_v, dst_v, rows_v, zero_v, acc_sh, gsems = args[2 * npass + 2:]
        c = lax.axis_index("c")
        s = lax.axis_index("s")
        w = c * NS + s
        pltpu.sync_copy(src_hbm.at[w], src_v)
        pltpu.sync_copy(dst_hbm.at[w], dst_v)

        def _gather(g_hbm, j, b):
            pltpu.async_copy(g_hbm.at[src_v.at[j]], rows_v.at[b], gsems[b])

        def _gather_wait(g_hbm, j, b):
            pltpu.make_async_copy(
                g_hbm.at[src_v.at[j]], rows_v.at[b], gsems[b]).wait()

        def _scatter_sync(j, b):
            pltpu.sync_copy(rows_v.at[b], acc_sh.at[dst_v.at[j]], add=True)

        def _zero_stripe():
            for p in range(STRIPE // K):
                pltpu.sync_copy(zero_v,
                                acc_sh.at[pl.ds(s * STRIPE + p * K, K)])

        zeros16 = jnp.zeros((16,), jnp.float32)

        def zrow(i, carry):
            r = i // (D // 16)
            l = i % (D // 16)
            zero_v[r, pl.ds(l * 16, 16)] = zeros16
            return carry

        for b in range(NBUF):
            _gather(g_hbms[0], b, b)
        lax.fori_loop(0, (K * D) // 16, zrow, 0)
        _zero_stripe()
        plsc.subcore_barrier()

        ngrp = J // NBUF
        for p in range(npass):
            g_hbm = g_hbms[p]
            nxt = g_hbms[p + 1] if p + 1 < npass else None

            def body(g, carry, g_hbm=g_hbm):
                for b in range(NBUF):
                    j = g * NBUF + b
                    _gather_wait(g_hbm, j, b)
                    _scatter_sync(j, b)
                    _gather(g_hbm, j + NBUF, b)
                return carry

            lax.fori_loop(0, ngrp - 1, body, 0)
            for b in range(NBUF):
                jt = (ngrp - 1) * NBUF + b
                _gather_wait(g_hbm, jt, b)
                _scatter_sync(jt, b)
                if nxt is not None:
                    _gather(nxt, b, b)
            plsc.subcore_barrier()
            pltpu.sync_copy(acc_sh.at[pl.ds(s * STRIPE, STRIPE)],
                            out_hbms[p].at[c, pl.ds(s * STRIPE, STRIPE)])
            if nxt is not None:
                _zero_stripe()
                plsc.subcore_barrier()

    return _scatter_kernel


_scatter_l1 = _make_scatter(D_OUT, 2)
_scatter_l2 = _make_scatter(D_OUT, 1)


_R = 1024


def _dinv_of(hist_blk):
    deg = jnp.sum(hist_blk, axis=1, keepdims=True) + 1.0
    return lax.rsqrt(deg)


def _tc1_body(hist_ref, x_ref, w1_ref, g1a_ref, g1b_ref, dinv_ref):
    dinv = _dinv_of(hist_ref[...])
    g1 = jnp.dot(x_ref[...], w1_ref[...],
                 preferred_element_type=jnp.float32) * dinv
    g1a_ref[...] = g1[:, :D_OUT]
    g1b_ref[...] = g1[:, D_OUT:]
    dinv_ref[...] = dinv


def _tc2_body(dinv_ref, acca_ref, accb_ref, g1a_ref, g1b_ref,
              b1a_ref, b1b_ref, w2a_ref, w2b_ref, g2_ref):
    dinv = dinv_ref[...]
    ha = jnp.maximum((acca_ref[0] + acca_ref[1] + g1a_ref[...]) * dinv
                     + b1a_ref[...], 0.0)
    hb = jnp.maximum((accb_ref[0] + accb_ref[1] + g1b_ref[...]) * dinv
                     + b1b_ref[...], 0.0)
    g2 = (jnp.dot(ha, w2a_ref[...], preferred_element_type=jnp.float32)
          + jnp.dot(hb, w2b_ref[...], preferred_element_type=jnp.float32))
    g2_ref[...] = g2 * dinv


def _tc3_body(dinv_ref, acc_ref, g2_ref, b2_ref, out_ref):
    dinv = dinv_ref[...]
    a = acc_ref[0] + acc_ref[1] + g2_ref[...]
    out_ref[...] = jnp.maximum(a * dinv + b2_ref[...], 0.0)


def _row_spec(d):
    return pl.BlockSpec((_R, d), lambda i: (i, 0))


def _acc_spec(d):
    return pl.BlockSpec((NC, _R, d), lambda i: (0, i, 0))


def _full_spec(r, d):
    return pl.BlockSpec((r, d), lambda i: (0, 0))


_GRID = N_PAD // _R

_tc1 = pl.pallas_call(
    _tc1_body,
    grid=(_GRID,),
    in_specs=[_row_spec(NW), _row_spec(D_IN), _full_spec(D_IN, D_HID)],
    out_specs=[_row_spec(D_OUT), _row_spec(D_OUT), _row_spec(1)],
    out_shape=[jax.ShapeDtypeStruct((N_PAD, D_OUT), jnp.float32),
               jax.ShapeDtypeStruct((N_PAD, D_OUT), jnp.float32),
               jax.ShapeDtypeStruct((N_PAD, 1), jnp.float32)],
)

_PK = N_PAD // 2
_PKGRID = _PK // _R

_tc2 = pl.pallas_call(
    _tc2_body,
    grid=(_PKGRID,),
    in_specs=[_row_spec(D_HID), _acc_spec(D_HID), _acc_spec(D_HID),
              _row_spec(D_HID), _row_spec(D_HID),
              _full_spec(1, D_HID), _full_spec(1, D_HID),
              _full_spec(D_HID, D_HID), _full_spec(D_HID, D_HID)],
    out_specs=_row_spec(D_HID),
    out_shape=jax.ShapeDtypeStruct((_PK, D_HID), jnp.float32),
)

_tc3 = pl.pallas_call(
    _tc3_body,
    grid=(_PKGRID,),
    in_specs=[_row_spec(D_HID), _acc_spec(D_HID), _row_spec(D_HID),
              _full_spec(1, D_HID)],
    out_specs=_row_spec(D_HID),
    out_shape=jax.ShapeDtypeStruct((N // 2, D_HID), jnp.float32),
)


def _edge_blocks(idx_flat):
    fill = N + (jnp.arange(E_PAD - E, dtype=jnp.int32) % (N_PAD - N))
    return jnp.concatenate([idx_flat, fill]).reshape(NW, J, K)


def kernel(x, edge_index, W1, b1, W2, b2):
    x_pad = jnp.zeros((N_PAD, D_IN), x.dtype).at[:N].set(x)
    src = _edge_blocks(edge_index[0])
    dst = _edge_blocks(edge_index[1])

    hist = _deg_kernel(dst)
    hist_t = hist.T

    g1a, g1b, dinv = _tc1(hist_t, x_pad, W1)

    pk = lambda a: a.reshape(_PK, D_HID)
    pk2 = lambda a: a.reshape(NC, _PK, D_HID)
    acc1a, acc1b = _scatter_l1(g1a, g1b, src, dst)
    dinv_pk = jnp.repeat(dinv[:, 0], D_OUT).reshape(_PK, D_HID)
    zblk = jnp.zeros((D_OUT, D_OUT), jnp.float32)
    w2a_bd = jnp.block([[W2[:D_OUT], zblk], [zblk, W2[:D_OUT]]])
    w2b_bd = jnp.block([[W2[D_OUT:], zblk], [zblk, W2[D_OUT:]]])
    b1a = jnp.tile(b1[:D_OUT], 2).reshape(1, D_HID)
    b1b = jnp.tile(b1[D_OUT:], 2).reshape(1, D_HID)
    b2pk = jnp.tile(b2, 2).reshape(1, D_HID)

    g2_pk = _tc2(dinv_pk, pk2(acc1a), pk2(acc1b), pk(g1a), pk(g1b),
                 b1a, b1b, w2a_bd, w2b_bd)
    g2 = g2_pk.reshape(N_PAD, D_OUT)
    acc2, = _scatter_l2(g2, src, dst)
    out_pk = _tc3(dinv_pk, pk2(acc2), g2_pk, b2pk)
    return out_pk.reshape(N, D_OUT)
